# baseline (speedup 1.0000x reference)
import jax
import jax.numpy as jnp
from jax import lax
from jax.experimental import pallas as pl
from jax.experimental.pallas import tpu as pltpu

N = 16384
FIN = 4
F = 16
K = 8
NB = 64
R = 256
C = 256
G = N // R
ER = 1024

INF = float('inf')
IMAX = 2**31 - 1
import numpy as _np
BIGP = (int(_np.float32(1e9).view(_np.int32)) & -256)
INFP = (int(_np.float32(_np.inf).view(_np.int32)) & -256)


def _elu(x):
    return jnp.where(x > 0, x, jnp.exp(x) - 1.0)


def _dot_t(a, b):
    return lax.dot_general(a, b, (((1,), (1,)), ((), ())),
                           preferred_element_type=jnp.float32)


def _encode_kernel(x_ref, w1_ref, b1_ref, w2_ref, b2_ref, wc_ref,
                   h_ref, ht_ref, sq_ref):
    h1 = _elu(_dot_t(x_ref[...], w1_ref[...]) + b1_ref[...])
    h = _elu(_dot_t(h1, w2_ref[...]) + b2_ref[...])
    h_ref[...] = h
    ht_ref[...] = _dot_t(h, wc_ref[:, F:])
    sq_ref[...] = jnp.sum(h * h, axis=1, keepdims=True)


def _knn_kernel(cs_ref, cn_ref, nb_ref, h_ref, ht_ref, sq_ref, sqt_ref,
                b_ref, bt_ref, wc_ref, bc_ref, sums_ref, cnts_ref):
    g = pl.program_id(0)
    r0 = g * R
    h_i = h_ref[pl.ds(r0, R), :]
    sq_i = sq_ref[pl.ds(r0, R), :]
    b_i = b_ref[pl.ds(r0, R), :]

    iota_ci = lax.broadcasted_iota(jnp.int32, (1, C), 1)
    hi2 = h_i * -2.0

    def tree(p_, leaves):
        b0 = (p_ & 1) > 0
        b1 = (p_ & 2) > 0
        b2 = (p_ & 4) > 0
        l0 = jnp.where(b0, leaves[1], leaves[0])
        l1 = jnp.where(b0, leaves[3], leaves[2])
        l2 = jnp.where(b0, leaves[5], leaves[4])
        l3 = jnp.where(b0, leaves[7], leaves[6])
        m0 = jnp.where(b1, l1, l0)
        m1 = jnp.where(b1, l3, l2)
        return jnp.where(b2, m1, m0)

    def merge(state, pk, ht_c):
        sd, best_f = state
        new_d, new_f = [], []
        for t in range(K):
            m_c = jnp.min(pk, axis=1, keepdims=True)
            m_s = jnp.min(sd, axis=1, keepdims=True)
            use_c = m_c < m_s
            new_d.append((jnp.minimum(m_c, m_s) & -256) | t)
            j_s = m_s & 255
            hit = (pk == m_c) & use_c
            f_c = lax.dot_general(jnp.where(hit, 1.0, 0.0), ht_c,
                                  (((1,), (0,)), ((), ())),
                                  preferred_element_type=jnp.float32)
            f_s = tree(j_s, best_f)
            new_f.append(jnp.where(use_c, f_c, f_s))
            pk = jnp.where(hit, IMAX, pk)
            sd = jnp.where((sd == m_s) & (~use_c), IMAX, sd)
        return jnp.concatenate(new_d, axis=1), tuple(new_f)

    def chunk_pk(col0):
        h_c = h_ref[pl.ds(col0, C), :]
        ht_c = ht_ref[pl.ds(col0, C), :]
        sq_c = sqt_ref[:, pl.ds(col0, C)]
        b_c = bt_ref[:, pl.ds(col0, C)]
        d2 = jnp.maximum((sq_i + sq_c) + _dot_t(hi2, h_c), 0.0)
        pk = (lax.bitcast_convert_type(d2, jnp.int32) & -256) | iota_ci
        return pk, (b_i == b_c), ht_c

    state = (jnp.full((R, K), IMAX, jnp.int32),
             tuple(jnp.zeros((R, F), jnp.float32) for _ in range(K)))

    def do_backfill(st):
        _pku, same0, ht_c0 = chunk_pk(0)
        return merge(st, jnp.where(same0, IMAX, BIGP | iota_ci), ht_c0)

    state = lax.cond(nb_ref[g] > 0, do_backfill, lambda st: st, state)

    cs = cs_ref[g]

    def body(j, st):
        col0 = (cs + j) * C
        pk, same, ht_c = chunk_pk(col0)
        return merge(st, jnp.where(same, pk, IMAX), ht_c)

    state = lax.fori_loop(0, cn_ref[g], body, state)
    _best_d, best_f = state

    c_i = _dot_t(h_i, wc_ref[:, :F] - wc_ref[:, F:]) + bc_ref[...]
    m0 = jnp.maximum(jnp.maximum(best_f[0], best_f[1]),
                     jnp.maximum(best_f[2], best_f[3]))
    m1 = jnp.maximum(jnp.maximum(best_f[4], best_f[5]),
                     jnp.maximum(best_f[6], best_f[7]))
    feats = _elu(c_i + jnp.maximum(m0, m1))

    iota_b = lax.broadcasted_iota(jnp.int32, (1, NB), 1)
    ohb = jnp.where(b_i == iota_b, 1.0, 0.0)
    part_sums = lax.dot_general(ohb, feats, (((0,), (0,)), ((), ())),
                                preferred_element_type=jnp.float32)
    part_cnts = lax.dot_general(ohb, jnp.ones((R, 1), jnp.float32),
                                (((0,), (0,)), ((), ())),
                                preferred_element_type=jnp.float32)

    @pl.when(g == 0)
    def _init():
        sums_ref[...] = jnp.zeros_like(sums_ref)
        cnts_ref[...] = jnp.zeros_like(cnts_ref)

    sums_ref[...] += part_sums
    cnts_ref[...] += part_cnts


def _head_kernel(sums_ref, cnts_ref, perm_ref, fill_ref, wo1_ref, bo1_ref,
                 wo2_ref, bo2_ref, wo3_ref, bo3_ref, out_ref):
    pooled_v = sums_ref[...] / jnp.maximum(cnts_ref[...], 1.0)
    pooled = lax.dot_general(perm_ref[...], pooled_v, (((1,), (0,)), ((), ())),
                             preferred_element_type=jnp.float32)
    pooled = jnp.where(fill_ref[...] > 0, float('nan'), pooled)
    o1 = _elu(_dot_t(pooled, wo1_ref[...]) + bo1_ref[...])
    o2 = _elu(_dot_t(o1, wo2_ref[...]) + bo2_ref[...])
    out_ref[...] = _dot_t(o2, wo3_ref[...]) + bo3_ref[...]


def kernel(x_pf, batch_pf, W1, b1, W2, b2, Wc, bc, Wo1, bo1, Wo2, bo2,
           Wo3, bo3):
    batch = batch_pf.astype(jnp.int32)

    h, ht, sq = pl.pallas_call(
        _encode_kernel,
        grid=(N // ER,),
        in_specs=[
            pl.BlockSpec((ER, FIN), lambda i: (i, 0)),
            pl.BlockSpec((F, FIN), lambda i: (0, 0)),
            pl.BlockSpec((1, F), lambda i: (0, 0)),
            pl.BlockSpec((F, F), lambda i: (0, 0)),
            pl.BlockSpec((1, F), lambda i: (0, 0)),
            pl.BlockSpec((F, 2 * F), lambda i: (0, 0)),
        ],
        out_specs=[
            pl.BlockSpec((ER, F), lambda i: (i, 0)),
            pl.BlockSpec((ER, F), lambda i: (i, 0)),
            pl.BlockSpec((ER, 1), lambda i: (i, 0)),
        ],
        out_shape=[
            jax.ShapeDtypeStruct((N, F), jnp.float32),
            jax.ShapeDtypeStruct((N, F), jnp.float32),
            jax.ShapeDtypeStruct((N, 1), jnp.float32),
        ],
    )(x_pf, W1, b1.reshape(1, F), W2, b2.reshape(1, F), Wc)

    offsets = jnp.searchsorted(batch, jnp.arange(NB + 1, dtype=jnp.int32),
                               side='left').astype(jnp.int32)
    b_first = batch[::R]
    b_last = batch[R - 1::R]
    col_lo = jnp.take(offsets, b_first)
    col_hi = jnp.take(offsets, b_last + 1)
    cs = col_lo // C
    cn = (col_hi + C - 1) // C - cs
    sizes = offsets[1:] - offsets[:-1]
    vals = jnp.arange(NB, dtype=jnp.int32)
    touched = (vals[None, :] >= b_first[:, None]) & (vals[None, :] <= b_last[:, None])
    min_sz = jnp.min(jnp.where(touched, sizes[None, :], N), axis=1)
    nb = (min_sz < K).astype(jnp.int32)

    sums, cnts = pl.pallas_call(
        _knn_kernel,
        grid_spec=pltpu.PrefetchScalarGridSpec(
            num_scalar_prefetch=3,
            grid=(G,),
            in_specs=[
                pl.BlockSpec((N, F), lambda g, *_: (0, 0)),
                pl.BlockSpec((N, F), lambda g, *_: (0, 0)),
                pl.BlockSpec((N, 1), lambda g, *_: (0, 0)),
                pl.BlockSpec((1, N), lambda g, *_: (0, 0)),
                pl.BlockSpec((N, 1), lambda g, *_: (0, 0)),
                pl.BlockSpec((1, N), lambda g, *_: (0, 0)),
                pl.BlockSpec((F, 2 * F), lambda g, *_: (0, 0)),
                pl.BlockSpec((1, F), lambda g, *_: (0, 0)),
            ],
            out_specs=[
                pl.BlockSpec((NB, F), lambda g, *_: (0, 0)),
                pl.BlockSpec((NB, 1), lambda g, *_: (0, 0)),
            ],
        ),
        out_shape=[
            jax.ShapeDtypeStruct((NB, F), jnp.float32),
            jax.ShapeDtypeStruct((NB, 1), jnp.float32),
        ],
    )(cs, cn, nb, h, ht, sq, sq.reshape(1, N), batch.reshape(N, 1),
      batch.reshape(1, N), Wc, bc.reshape(1, F))

    sizes = offsets[1:] - offsets[:-1]
    present = sizes > 0
    ranks = jnp.cumsum(present.astype(jnp.int32)) - 1
    vals = jnp.arange(NB, dtype=jnp.int32)
    uniq = jnp.zeros((NB,), jnp.int32).at[
        jnp.where(present, ranks, NB)].set(vals, mode='drop')
    perm = ((ranks[None, :] == vals[:, None]) & present[None, :]
            ).astype(jnp.float32)
    fill = (vals >= jnp.sum(present.astype(jnp.int32))
            ).astype(jnp.float32).reshape(NB, 1)

    out = pl.pallas_call(
        _head_kernel,
        out_shape=jax.ShapeDtypeStruct((NB, 2), jnp.float32),
    )(sums, cnts, perm, fill, Wo1, bo1.reshape(1, 8), Wo2, bo2.reshape(1, 4),
      Wo3, bo3.reshape(1, 2))

    return (out, uniq.astype(batch_pf.dtype))


# speedup vs baseline: 2.1453x; 2.1453x over previous
import jax
import jax.numpy as jnp
from jax import lax
from jax.experimental import pallas as pl
from jax.experimental.pallas import tpu as pltpu

N = 16384
FIN = 4
F = 16
K = 8
NB = 64
R = 256
C = 256
G = N // R
ER = 1024

INF = float('inf')
IMAX = 2**31 - 1
import numpy as _np
BIGP = (int(_np.float32(1e9).view(_np.int32)) & -256)
INFP = (int(_np.float32(_np.inf).view(_np.int32)) & -256)


def _elu(x):
    return jnp.where(x > 0, x, jnp.exp(x) - 1.0)


def _dot_t(a, b):
    return lax.dot_general(a, b, (((1,), (1,)), ((), ())),
                           preferred_element_type=jnp.float32)


def _encode_kernel(x_ref, w1_ref, b1_ref, w2_ref, b2_ref, wc_ref,
                   h_ref, ht_ref, sq_ref):
    h1 = _elu(_dot_t(x_ref[...], w1_ref[...]) + b1_ref[...])
    h = _elu(_dot_t(h1, w2_ref[...]) + b2_ref[...])
    h_ref[...] = h
    ht_ref[...] = _dot_t(h, wc_ref[:, F:])
    sq_ref[...] = jnp.sum(h * h, axis=1, keepdims=True)


def _knn_kernel(cs_ref, cn_ref, nb_ref, h_ref, ht_ref, sq_ref, sqt_ref,
                b_ref, bt_ref, wc_ref, bc_ref, sums_ref, cnts_ref):
    g = pl.program_id(0)
    r0 = g * R
    h_i = h_ref[pl.ds(r0, R), :]
    sq_it = sqt_ref[:, pl.ds(r0, R)]
    b_it = bt_ref[:, pl.ds(r0, R)]

    iota_cs = lax.broadcasted_iota(jnp.int32, (C, 1), 0)
    hi2 = h_i * -2.0

    def tree(p_, leaves):
        b0 = (p_ & 1) > 0
        b1 = (p_ & 2) > 0
        b2 = (p_ & 4) > 0
        l0 = jnp.where(b0, leaves[1], leaves[0])
        l1 = jnp.where(b0, leaves[3], leaves[2])
        l2 = jnp.where(b0, leaves[5], leaves[4])
        l3 = jnp.where(b0, leaves[7], leaves[6])
        m0 = jnp.where(b1, l1, l0)
        m1 = jnp.where(b1, l3, l2)
        return jnp.where(b2, m1, m0)

    def merge(state, pk, ht_c):
        sd, best_f = state
        new_d, new_f = [], []
        for t in range(K):
            m_c = jnp.min(pk, axis=0, keepdims=True)
            a0 = jnp.minimum(sd[0], sd[1])
            a1 = jnp.minimum(sd[2], sd[3])
            a2 = jnp.minimum(sd[4], sd[5])
            a3 = jnp.minimum(sd[6], sd[7])
            m_s = jnp.minimum(jnp.minimum(a0, a1), jnp.minimum(a2, a3))
            use_c = m_c < m_s
            new_d.append((jnp.minimum(m_c, m_s) & -256) | t)
            j_s = m_s & 255
            hit = (pk == m_c) & use_c
            f_c = lax.dot_general(ht_c, jnp.where(hit, 1.0, 0.0),
                                  (((0,), (0,)), ((), ())),
                                  preferred_element_type=jnp.float32)
            f_s = tree(j_s, best_f)
            new_f.append(jnp.where(use_c, f_c, f_s))
            pk = jnp.where(hit, IMAX, pk)
            sd = tuple(jnp.where((sd[s] == m_s) & (~use_c), IMAX, sd[s])
                       for s in range(K))
        return tuple(new_d), tuple(new_f)

    def chunk_pk(col0):
        h_c = h_ref[pl.ds(col0, C), :]
        ht_c = ht_ref[pl.ds(col0, C), :]
        sq_c = sq_ref[pl.ds(col0, C), :]
        b_c = b_ref[pl.ds(col0, C), :]
        d2 = jnp.maximum((sq_c + sq_it) + lax.dot_general(
            h_c, hi2, (((1,), (1,)), ((), ())),
            preferred_element_type=jnp.float32), 0.0)
        pk = (lax.bitcast_convert_type(d2, jnp.int32) & -256) | iota_cs
        return pk, (b_c == b_it), ht_c

    state = (tuple(jnp.full((1, R), IMAX, jnp.int32) for _ in range(K)),
             tuple(jnp.zeros((F, R), jnp.float32) for _ in range(K)))

    def do_backfill(st):
        _pku, same0, ht_c0 = chunk_pk(0)
        return merge(st, jnp.where(same0, IMAX, BIGP | iota_cs), ht_c0)

    state = lax.cond(nb_ref[g] > 0, do_backfill, lambda st: st, state)

    cs = cs_ref[g]

    def body(j, st):
        col0 = (cs + j) * C
        pk, same, ht_c = chunk_pk(col0)
        return merge(st, jnp.where(same, pk, IMAX), ht_c)

    state = lax.fori_loop(0, cn_ref[g], body, state)
    _best_d, best_f = state

    c_i = lax.dot_general(wc_ref[:, :F] - wc_ref[:, F:], h_i,
                          (((1,), (1,)), ((), ())),
                          preferred_element_type=jnp.float32) + bc_ref[...]
    m0 = jnp.maximum(jnp.maximum(best_f[0], best_f[1]),
                     jnp.maximum(best_f[2], best_f[3]))
    m1 = jnp.maximum(jnp.maximum(best_f[4], best_f[5]),
                     jnp.maximum(best_f[6], best_f[7]))
    feats = _elu(c_i + jnp.maximum(m0, m1))

    iota_b = lax.broadcasted_iota(jnp.int32, (NB, 1), 0)
    ohb = jnp.where(b_it == iota_b, 1.0, 0.0)
    part_sums = lax.dot_general(ohb, feats, (((1,), (1,)), ((), ())),
                                preferred_element_type=jnp.float32)
    part_cnts = lax.dot_general(ohb, jnp.ones((1, R), jnp.float32),
                                (((1,), (1,)), ((), ())),
                                preferred_element_type=jnp.float32)

    @pl.when(g == 0)
    def _init():
        sums_ref[...] = jnp.zeros_like(sums_ref)
        cnts_ref[...] = jnp.zeros_like(cnts_ref)

    sums_ref[...] += part_sums
    cnts_ref[...] += part_cnts


def _head_kernel(sums_ref, cnts_ref, perm_ref, fill_ref, wo1_ref, bo1_ref,
                 wo2_ref, bo2_ref, wo3_ref, bo3_ref, out_ref):
    pooled_v = sums_ref[...] / jnp.maximum(cnts_ref[...], 1.0)
    pooled = lax.dot_general(perm_ref[...], pooled_v, (((1,), (0,)), ((), ())),
                             preferred_element_type=jnp.float32)
    pooled = jnp.where(fill_ref[...] > 0, float('nan'), pooled)
    o1 = _elu(_dot_t(pooled, wo1_ref[...]) + bo1_ref[...])
    o2 = _elu(_dot_t(o1, wo2_ref[...]) + bo2_ref[...])
    out_ref[...] = _dot_t(o2, wo3_ref[...]) + bo3_ref[...]


def kernel(x_pf, batch_pf, W1, b1, W2, b2, Wc, bc, Wo1, bo1, Wo2, bo2,
           Wo3, bo3):
    batch = batch_pf.astype(jnp.int32)

    h, ht, sq = pl.pallas_call(
        _encode_kernel,
        grid=(N // ER,),
        in_specs=[
            pl.BlockSpec((ER, FIN), lambda i: (i, 0)),
            pl.BlockSpec((F, FIN), lambda i: (0, 0)),
            pl.BlockSpec((1, F), lambda i: (0, 0)),
            pl.BlockSpec((F, F), lambda i: (0, 0)),
            pl.BlockSpec((1, F), lambda i: (0, 0)),
            pl.BlockSpec((F, 2 * F), lambda i: (0, 0)),
        ],
        out_specs=[
            pl.BlockSpec((ER, F), lambda i: (i, 0)),
            pl.BlockSpec((ER, F), lambda i: (i, 0)),
            pl.BlockSpec((ER, 1), lambda i: (i, 0)),
        ],
        out_shape=[
            jax.ShapeDtypeStruct((N, F), jnp.float32),
            jax.ShapeDtypeStruct((N, F), jnp.float32),
            jax.ShapeDtypeStruct((N, 1), jnp.float32),
        ],
    )(x_pf, W1, b1.reshape(1, F), W2, b2.reshape(1, F), Wc)

    offsets = jnp.searchsorted(batch, jnp.arange(NB + 1, dtype=jnp.int32),
                               side='left').astype(jnp.int32)
    b_first = batch[::R]
    b_last = batch[R - 1::R]
    col_lo = jnp.take(offsets, b_first)
    col_hi = jnp.take(offsets, b_last + 1)
    cs = col_lo // C
    cn = (col_hi + C - 1) // C - cs
    sizes = offsets[1:] - offsets[:-1]
    vals = jnp.arange(NB, dtype=jnp.int32)
    touched = (vals[None, :] >= b_first[:, None]) & (vals[None, :] <= b_last[:, None])
    min_sz = jnp.min(jnp.where(touched, sizes[None, :], N), axis=1)
    nb = (min_sz < K).astype(jnp.int32)

    sums, cnts = pl.pallas_call(
        _knn_kernel,
        grid_spec=pltpu.PrefetchScalarGridSpec(
            num_scalar_prefetch=3,
            grid=(G,),
            in_specs=[
                pl.BlockSpec((N, F), lambda g, *_: (0, 0)),
                pl.BlockSpec((N, F), lambda g, *_: (0, 0)),
                pl.BlockSpec((N, 1), lambda g, *_: (0, 0)),
                pl.BlockSpec((1, N), lambda g, *_: (0, 0)),
                pl.BlockSpec((N, 1), lambda g, *_: (0, 0)),
                pl.BlockSpec((1, N), lambda g, *_: (0, 0)),
                pl.BlockSpec((F, 2 * F), lambda g, *_: (0, 0)),
                pl.BlockSpec((F, 1), lambda g, *_: (0, 0)),
            ],
            out_specs=[
                pl.BlockSpec((NB, F), lambda g, *_: (0, 0)),
                pl.BlockSpec((NB, 1), lambda g, *_: (0, 0)),
            ],
        ),
        out_shape=[
            jax.ShapeDtypeStruct((NB, F), jnp.float32),
            jax.ShapeDtypeStruct((NB, 1), jnp.float32),
        ],
    )(cs, cn, nb, h, ht, sq, sq.reshape(1, N), batch.reshape(N, 1),
      batch.reshape(1, N), Wc, bc.reshape(F, 1))

    sizes = offsets[1:] - offsets[:-1]
    present = sizes > 0
    ranks = jnp.cumsum(present.astype(jnp.int32)) - 1
    vals = jnp.arange(NB, dtype=jnp.int32)
    uniq = jnp.zeros((NB,), jnp.int32).at[
        jnp.where(present, ranks, NB)].set(vals, mode='drop')
    perm = ((ranks[None, :] == vals[:, None]) & present[None, :]
            ).astype(jnp.float32)
    fill = (vals >= jnp.sum(present.astype(jnp.int32))
            ).astype(jnp.float32).reshape(NB, 1)

    out = pl.pallas_call(
        _head_kernel,
        out_shape=jax.ShapeDtypeStruct((NB, 2), jnp.float32),
    )(sums, cnts, perm, fill, Wo1, bo1.reshape(1, 8), Wo2, bo2.reshape(1, 4),
      Wo3, bo3.reshape(1, 2))

    return (out, uniq.astype(batch_pf.dtype))


# speedup vs baseline: 2.1466x; 1.0006x over previous
import jax
import jax.numpy as jnp
from jax import lax
from jax.experimental import pallas as pl
from jax.experimental.pallas import tpu as pltpu

N = 16384
FIN = 4
F = 16
K = 8
NB = 64
R = 256
C = 256
G = N // R
ER = 1024

IMAX = 2**31 - 1
import numpy as _np
BIGP = (int(_np.float32(1e9).view(_np.int32)) & -256)


def _elu(x):
    return jnp.where(x > 0, x, jnp.exp(x) - 1.0)


def _dot_t(a, b):
    return lax.dot_general(a, b, (((1,), (1,)), ((), ())),
                           preferred_element_type=jnp.float32)


def _encode_kernel(x_ref, w1_ref, b1_ref, w2_ref, b2_ref, wc_ref,
                   h_ref, ht_ref, sq_ref):
    h1 = _elu(_dot_t(x_ref[...], w1_ref[...]) + b1_ref[...])
    h = _elu(_dot_t(h1, w2_ref[...]) + b2_ref[...])
    h_ref[...] = h
    ht_ref[...] = _dot_t(h, wc_ref[:, F:])
    sq_ref[...] = jnp.sum(h * h, axis=1, keepdims=True)


def _knn_kernel(cs_ref, cn_ref, nb_ref, h_ref, ht_ref, sq_ref, sqt_ref,
                b_ref, bt_ref, wc_ref, bc_ref, sums_ref, cnts_ref):
    g = pl.program_id(0)
    r0 = g * R
    h_i = h_ref[pl.ds(r0, R), :]
    sq_it = sqt_ref[:, pl.ds(r0, R)]
    b_it = bt_ref[:, pl.ds(r0, R)]

    iota_cs = lax.broadcasted_iota(jnp.int32, (C, 1), 0)
    hi2 = h_i * -2.0

    def tree(p_, leaves):
        b0 = (p_ & 1) > 0
        b1 = (p_ & 2) > 0
        b2 = (p_ & 4) > 0
        l0 = jnp.where(b0, leaves[1], leaves[0])
        l1 = jnp.where(b0, leaves[3], leaves[2])
        l2 = jnp.where(b0, leaves[5], leaves[4])
        l3 = jnp.where(b0, leaves[7], leaves[6])
        m0 = jnp.where(b1, l1, l0)
        m1 = jnp.where(b1, l3, l2)
        return jnp.where(b2, m1, m0)

    def merge(state, pk, ht_c):
        sd, best_f = state
        new_d, new_f = [], []
        for t in range(K):
            m_c = jnp.min(pk, axis=0, keepdims=True)
            a0 = jnp.minimum(sd[0], sd[1])
            a1 = jnp.minimum(sd[2], sd[3])
            a2 = jnp.minimum(sd[4], sd[5])
            a3 = jnp.minimum(sd[6], sd[7])
            m_s = jnp.minimum(jnp.minimum(a0, a1), jnp.minimum(a2, a3))
            use_c = m_c < m_s
            new_d.append((jnp.minimum(m_c, m_s) & -256) | t)
            j_s = m_s & 255
            hit = (pk == m_c) & use_c
            f_c = lax.dot_general(ht_c, jnp.where(hit, 1.0, 0.0),
                                  (((0,), (0,)), ((), ())),
                                  preferred_element_type=jnp.float32)
            f_s = tree(j_s, best_f)
            new_f.append(jnp.where(use_c, f_c, f_s))
            pk = jnp.where(hit, IMAX, pk)
            sd = tuple(jnp.where((sd[s] == m_s) & (~use_c), IMAX, sd[s])
                       for s in range(K))
        return tuple(new_d), tuple(new_f)

    def chunk_pk(col0):
        h_c = h_ref[pl.ds(col0, C), :]
        ht_c = ht_ref[pl.ds(col0, C), :]
        sq_c = sq_ref[pl.ds(col0, C), :]
        b_c = b_ref[pl.ds(col0, C), :]
        d2 = jnp.maximum((sq_c + sq_it) + lax.dot_general(
            h_c, hi2, (((1,), (1,)), ((), ())),
            preferred_element_type=jnp.float32), 0.0)
        pk = (lax.bitcast_convert_type(d2, jnp.int32) & -256) | iota_cs
        return pk, (b_c == b_it), ht_c

    state = (tuple(jnp.full((1, R), IMAX, jnp.int32) for _ in range(K)),
             tuple(jnp.zeros((F, R), jnp.float32) for _ in range(K)))

    def do_backfill(st):
        _pku, same0, ht_c0 = chunk_pk(0)
        return merge(st, jnp.where(same0, IMAX, BIGP | iota_cs), ht_c0)

    state = lax.cond(nb_ref[g] > 0, do_backfill, lambda st: st, state)

    cs = cs_ref[g]

    def body(j, st):
        col0 = (cs + j) * C
        pk, same, ht_c = chunk_pk(col0)
        return merge(st, jnp.where(same, pk, IMAX), ht_c)

    state = lax.fori_loop(0, cn_ref[g], body, state)
    _best_d, best_f = state

    c_i = lax.dot_general(wc_ref[:, :F] - wc_ref[:, F:], h_i,
                          (((1,), (1,)), ((), ())),
                          preferred_element_type=jnp.float32) + bc_ref[...]
    m0 = jnp.maximum(jnp.maximum(best_f[0], best_f[1]),
                     jnp.maximum(best_f[2], best_f[3]))
    m1 = jnp.maximum(jnp.maximum(best_f[4], best_f[5]),
                     jnp.maximum(best_f[6], best_f[7]))
    feats = _elu(c_i + jnp.maximum(m0, m1))

    iota_b = lax.broadcasted_iota(jnp.int32, (NB, 1), 0)
    ohb = jnp.where(b_it == iota_b, 1.0, 0.0)
    part_sums = lax.dot_general(ohb, feats, (((1,), (1,)), ((), ())),
                                preferred_element_type=jnp.float32)
    part_cnts = lax.dot_general(ohb, jnp.ones((1, R), jnp.float32),
                                (((1,), (1,)), ((), ())),
                                preferred_element_type=jnp.float32)

    @pl.when(g == 0)
    def _init():
        sums_ref[...] = jnp.zeros_like(sums_ref)
        cnts_ref[...] = jnp.zeros_like(cnts_ref)

    sums_ref[...] += part_sums
    cnts_ref[...] += part_cnts


def _head_kernel(sums_ref, cnts_ref, perm_ref, fill_ref, wo1_ref, bo1_ref,
                 wo2_ref, bo2_ref, wo3_ref, bo3_ref, out_ref):
    pooled_v = sums_ref[...] / jnp.maximum(cnts_ref[...], 1.0)
    pooled = lax.dot_general(perm_ref[...], pooled_v, (((1,), (0,)), ((), ())),
                             preferred_element_type=jnp.float32)
    pooled = jnp.where(fill_ref[...] > 0, float('nan'), pooled)
    o1 = _elu(_dot_t(pooled, wo1_ref[...]) + bo1_ref[...])
    o2 = _elu(_dot_t(o1, wo2_ref[...]) + bo2_ref[...])
    out_ref[...] = _dot_t(o2, wo3_ref[...]) + bo3_ref[...]


def kernel(x_pf, batch_pf, W1, b1, W2, b2, Wc, bc, Wo1, bo1, Wo2, bo2,
           Wo3, bo3):
    batch = batch_pf.astype(jnp.int32)

    h, ht, sq = pl.pallas_call(
        _encode_kernel,
        grid=(N // ER,),
        in_specs=[
            pl.BlockSpec((ER, FIN), lambda i: (i, 0)),
            pl.BlockSpec((F, FIN), lambda i: (0, 0)),
            pl.BlockSpec((1, F), lambda i: (0, 0)),
            pl.BlockSpec((F, F), lambda i: (0, 0)),
            pl.BlockSpec((1, F), lambda i: (0, 0)),
            pl.BlockSpec((F, 2 * F), lambda i: (0, 0)),
        ],
        out_specs=[
            pl.BlockSpec((ER, F), lambda i: (i, 0)),
            pl.BlockSpec((ER, F), lambda i: (i, 0)),
            pl.BlockSpec((ER, 1), lambda i: (i, 0)),
        ],
        out_shape=[
            jax.ShapeDtypeStruct((N, F), jnp.float32),
            jax.ShapeDtypeStruct((N, F), jnp.float32),
            jax.ShapeDtypeStruct((N, 1), jnp.float32),
        ],
    )(x_pf, W1, b1.reshape(1, F), W2, b2.reshape(1, F), Wc)

    offsets = jnp.searchsorted(batch, jnp.arange(NB + 1, dtype=jnp.int32),
                               side='left').astype(jnp.int32)
    b_first = batch[::R]
    b_last = batch[R - 1::R]
    col_lo = jnp.take(offsets, b_first)
    col_hi = jnp.take(offsets, b_last + 1)
    cs = col_lo // C
    cn = (col_hi + C - 1) // C - cs
    sizes = offsets[1:] - offsets[:-1]
    vals = jnp.arange(NB, dtype=jnp.int32)
    touched = (vals[None, :] >= b_first[:, None]) & (vals[None, :] <= b_last[:, None])
    min_sz = jnp.min(jnp.where(touched, sizes[None, :], N), axis=1)
    nb = (min_sz < K).astype(jnp.int32)

    sums, cnts = pl.pallas_call(
        _knn_kernel,
        grid_spec=pltpu.PrefetchScalarGridSpec(
            num_scalar_prefetch=3,
            grid=(G,),
            in_specs=[
                pl.BlockSpec((N, F), lambda g, *_: (0, 0)),
                pl.BlockSpec((N, F), lambda g, *_: (0, 0)),
                pl.BlockSpec((N, 1), lambda g, *_: (0, 0)),
                pl.BlockSpec((1, N), lambda g, *_: (0, 0)),
                pl.BlockSpec((N, 1), lambda g, *_: (0, 0)),
                pl.BlockSpec((1, N), lambda g, *_: (0, 0)),
                pl.BlockSpec((F, 2 * F), lambda g, *_: (0, 0)),
                pl.BlockSpec((F, 1), lambda g, *_: (0, 0)),
            ],
            out_specs=[
                pl.BlockSpec((NB, F), lambda g, *_: (0, 0)),
                pl.BlockSpec((NB, 1), lambda g, *_: (0, 0)),
            ],
        ),
        out_shape=[
            jax.ShapeDtypeStruct((NB, F), jnp.float32),
            jax.ShapeDtypeStruct((NB, 1), jnp.float32),
        ],
    )(cs, cn, nb, h, ht, sq, sq.reshape(1, N), batch.reshape(N, 1),
      batch.reshape(1, N), Wc, bc.reshape(F, 1))

    sizes = offsets[1:] - offsets[:-1]
    present = sizes > 0
    ranks = jnp.cumsum(present.astype(jnp.int32)) - 1
    vals = jnp.arange(NB, dtype=jnp.int32)
    uniq = jnp.zeros((NB,), jnp.int32).at[
        jnp.where(present, ranks, NB)].set(vals, mode='drop')
    perm = ((ranks[None, :] == vals[:, None]) & present[None, :]
            ).astype(jnp.float32)
    fill = (vals >= jnp.sum(present.astype(jnp.int32))
            ).astype(jnp.float32).reshape(NB, 1)

    out = pl.pallas_call(
        _head_kernel,
        out_shape=jax.ShapeDtypeStruct((NB, 2), jnp.float32),
    )(sums, cnts, perm, fill, Wo1, bo1.reshape(1, 8), Wo2, bo2.reshape(1, 4),
      Wo3, bo3.reshape(1, 2))

    return (out, uniq.astype(batch_pf.dtype))
